# baseline (device time: 15578 ns/iter reference)
import jax
import jax.numpy as jnp
from jax import lax
from jax.experimental import pallas as pl
from jax.experimental.pallas import tpu as pltpu

N_DEV = 4


def kernel(A, B):
    M, K = A.shape
    K2, N = B.shape
    CH = M // N_DEV

    def body(a_hbm, b_hbm, out_hbm, a_v, b16, sbuf, rbuf, acc_v,
             send_sems, recv_sems, load_sems, store_sem):
        p = lax.axis_index("i")

        cp_a = pltpu.make_async_copy(a_hbm, a_v, load_sems.at[0])
        cp_a.start()
        cp_b = pltpu.make_async_copy(b_hbm, acc_v, load_sems.at[1])
        cp_b.start()

        barrier_sem = pltpu.get_barrier_semaphore()
        for k in range(1, N_DEV):
            pl.semaphore_signal(
                barrier_sem, inc=1,
                device_id=((p + k) % N_DEV,),
                device_id_type=pl.DeviceIdType.MESH,
            )

        cp_b.wait()
        b16[...] = acc_v[pl.ds(0, K), :].astype(jnp.bfloat16)
        cp_a.wait()

        def partial_chunk(q):
            return jnp.dot(
                a_v[pl.ds(q * CH, CH), :].astype(jnp.bfloat16), b16[...],
                preferred_element_type=jnp.float32,
            )

        order = (2, 1, 3)
        rdmas = []
        for j, k in enumerate(order):
            q = (p + k) % N_DEV
            sbuf[k - 1] = partial_chunk(q).astype(jnp.bfloat16)
            if j == 0:
                pl.semaphore_wait(barrier_sem, N_DEV - 1)
            rdma = pltpu.make_async_remote_copy(
                src_ref=sbuf.at[k - 1],
                dst_ref=rbuf.at[N_DEV - 1 - k],
                send_sem=send_sems.at[k - 1],
                recv_sem=recv_sems.at[N_DEV - 1 - k],
                device_id=(q,),
                device_id_type=pl.DeviceIdType.MESH,
            )
            rdma.start()
            rdmas.append(rdma)

        acc = partial_chunk(p)
        for k, rdma in zip(order, rdmas):
            rdma.wait()
            acc = acc + rbuf[N_DEV - 1 - k].astype(jnp.float32)

        acc_v[pl.ds(0, CH), :] = acc
        st = pltpu.make_async_copy(acc_v.at[pl.ds(0, CH), :], out_hbm, store_sem)
        st.start()
        st.wait()

    return pl.pallas_call(
        body,
        out_shape=jax.ShapeDtypeStruct((CH, N), jnp.float32),
        in_specs=[
            pl.BlockSpec(memory_space=pltpu.MemorySpace.HBM),
            pl.BlockSpec(memory_space=pltpu.MemorySpace.HBM),
        ],
        out_specs=pl.BlockSpec(memory_space=pltpu.MemorySpace.HBM),
        scratch_shapes=[
            pltpu.VMEM((M, K), jnp.float32),
            pltpu.VMEM((K, N), jnp.bfloat16),
            pltpu.VMEM((N_DEV - 1, CH, N), jnp.bfloat16),
            pltpu.VMEM((N_DEV - 1, CH, N), jnp.bfloat16),
            pltpu.VMEM((K, N), jnp.float32),
            pltpu.SemaphoreType.DMA((N_DEV - 1,)),
            pltpu.SemaphoreType.DMA((N_DEV - 1,)),
            pltpu.SemaphoreType.DMA((2,)),
            pltpu.SemaphoreType.DMA,
        ],
        compiler_params=pltpu.CompilerParams(collective_id=0),
    )(A, B)


# device time: 14833 ns/iter; 1.0502x vs baseline; 1.0502x over previous
import jax
import jax.numpy as jnp
from jax import lax
from jax.experimental import pallas as pl
from jax.experimental.pallas import tpu as pltpu

N_DEV = 4
HALVES = 2


def kernel(A, B):
    M, K = A.shape
    K2, N = B.shape
    CH = M // N_DEV
    H = CH // HALVES

    def body(a_ref, b_ref, out_ref, b16, sbuf, rbuf, send_sems, recv_sems):
        p = lax.axis_index("i")

        barrier_sem = pltpu.get_barrier_semaphore()
        for k in range(1, N_DEV):
            pl.semaphore_signal(
                barrier_sem, inc=1,
                device_id=((p + k) % N_DEV,),
                device_id_type=pl.DeviceIdType.MESH,
            )

        b16[...] = b_ref[...].astype(jnp.bfloat16)

        def partial_rows(row0):
            return jnp.dot(
                a_ref[pl.ds(row0, H), :].astype(jnp.bfloat16), b16[...],
                preferred_element_type=jnp.float32,
            )

        first = True
        rdmas = []
        for k in (2, 1, 3):
            q = (p + k) % N_DEV
            for h in range(HALVES):
                s = (k - 1) * HALVES + h
                r = (N_DEV - 1 - k) * HALVES + h
                sbuf[s] = partial_rows(q * CH + h * H).astype(jnp.bfloat16)
                if first:
                    pl.semaphore_wait(barrier_sem, N_DEV - 1)
                    first = False
                rdma = pltpu.make_async_remote_copy(
                    src_ref=sbuf.at[s],
                    dst_ref=rbuf.at[r],
                    send_sem=send_sems.at[s],
                    recv_sem=recv_sems.at[r],
                    device_id=(q,),
                    device_id_type=pl.DeviceIdType.MESH,
                )
                rdma.start()
                rdmas.append((k, h, rdma))

        acc = [partial_rows(p * CH + h * H) for h in range(HALVES)]
        for k, h, rdma in rdmas:
            rdma.wait()
            r = (N_DEV - 1 - k) * HALVES + h
            acc[h] = acc[h] + rbuf[r].astype(jnp.float32)
        for h in range(HALVES):
            out_ref[pl.ds(h * H, H), :] = acc[h]

    return pl.pallas_call(
        body,
        out_shape=jax.ShapeDtypeStruct((CH, N), jnp.float32),
        in_specs=[
            pl.BlockSpec(memory_space=pltpu.VMEM),
            pl.BlockSpec(memory_space=pltpu.VMEM),
        ],
        out_specs=pl.BlockSpec(memory_space=pltpu.VMEM),
        scratch_shapes=[
            pltpu.VMEM((K, N), jnp.bfloat16),
            pltpu.VMEM(((N_DEV - 1) * HALVES, H, N), jnp.bfloat16),
            pltpu.VMEM(((N_DEV - 1) * HALVES, H, N), jnp.bfloat16),
            pltpu.SemaphoreType.DMA(((N_DEV - 1) * HALVES,)),
            pltpu.SemaphoreType.DMA(((N_DEV - 1) * HALVES,)),
        ],
        compiler_params=pltpu.CompilerParams(collective_id=0),
    )(A, B)
